# baseline (speedup 1.0000x reference)
import functools

import jax
import jax.numpy as jnp
from jax import lax
from jax.experimental import pallas as pl
from jax.experimental.pallas import tpu as pltpu
from jax.experimental.pallas import tpu_sc as plsc

NC = 2
NS = 16
L = 16
NW = NC * NS

B = 16384
D = 32
RPS = 128 // D
BPW = B // NW
CH = 128
NCHUNK = BPW // CH

_mesh = plsc.VectorSubcoreMesh(core_axis_name="c", subcore_axis_name="s")


@functools.partial(
    pl.kernel,
    out_type=jax.ShapeDtypeStruct((B,), jnp.float32),
    mesh=_mesh,
    scratch_types=[
        pltpu.VMEM((BPW,), jnp.int32),
        pltpu.VMEM((BPW,), jnp.int32),
        pltpu.VMEM((BPW,), jnp.int32),
        pltpu.VMEM((BPW,), jnp.int32),
        pltpu.VMEM((2, CH, 128), jnp.float32),
        pltpu.VMEM((2, CH, 128), jnp.float32),
        pltpu.VMEM((2 * D * L,), jnp.float32),
        pltpu.VMEM((L,), jnp.float32),
        pltpu.VMEM((BPW,), jnp.float32),
        pltpu.SemaphoreType.DMA,
        pltpu.SemaphoreType.DMA,
        pltpu.SemaphoreType.DMA,
        pltpu.SemaphoreType.DMA,
    ],
    compiler_params=pltpu.CompilerParams(needs_layout_passes=False),
)
def _ncf_sc(uids, iids, su, si, utab, itab, wb, bb, out,
            uidx_v, iidx_v, su_v, si_v, ubuf, ibuf, w_v, b_v, out_v,
            sem_u0, sem_u1, sem_i0, sem_i1):
    wid = lax.axis_index("s") * NC + lax.axis_index("c")
    base = wid * BPW

    pltpu.sync_copy(su.at[pl.ds(base, BPW)], su_v)
    pltpu.sync_copy(si.at[pl.ds(base, BPW)], si_v)
    usems = (sem_u0, sem_u1)
    isems = (sem_i0, sem_i1)

    def gather(c):
        s = c % 2
        cu = pltpu.async_copy(
            utab.at[su_v.at[pl.ds(c * CH, CH)]], ubuf.at[s], usems[s])
        ci = pltpu.async_copy(
            itab.at[si_v.at[pl.ds(c * CH, CH)]], ibuf.at[s], isems[s])
        return cu, ci

    pend = gather(0)
    pltpu.sync_copy(uids.at[pl.ds(base, BPW)], uidx_v)
    pltpu.sync_copy(iids.at[pl.ds(base, BPW)], iidx_v)
    pltpu.sync_copy(wb, w_v)
    pltpu.sync_copy(bb, b_v)

    for c in range(NCHUNK):
        cu, ci = pend
        cu.wait()
        ci.wait()
        if c + 1 < NCHUNK:
            pend = gather(c + 1)
        ub = ubuf.at[c % 2]
        ib = ibuf.at[c % 2]

        def group(g, carry):
            r0 = c * CH + g * L
            rowi = g * L + lax.iota(jnp.int32, L)
            offu = (uidx_v[pl.ds(r0, L)] & (RPS - 1)) * D
            offi = (iidx_v[pl.ds(r0, L)] & (RPS - 1)) * D
            acc = b_v[...]
            for d in range(D):
                acc = acc + (plsc.load_gather(ub, [rowi, offu + d])
                             * w_v[pl.ds(d * L, L)])
            for d in range(D):
                acc = acc + (plsc.load_gather(ib, [rowi, offi + d])
                             * w_v[pl.ds((D + d) * L, L)])
            out_v[pl.ds(r0, L)] = acc
            return carry

        lax.fori_loop(0, CH // L, group, 0)

    pltpu.sync_copy(out_v, out.at[pl.ds(base, BPW)])


def kernel(user_ids, item_ids, user_table, item_table, W, b):
    su = lax.shift_right_logical(user_ids, 2)
    si = lax.shift_right_logical(item_ids, 2)
    ut = user_table.reshape(-1, 128)
    it = item_table.reshape(-1, 128)
    wb = jnp.broadcast_to(W.reshape(2 * D, 1), (2 * D, L)).reshape(-1)
    bb = jnp.broadcast_to(b, (L,))
    out = _ncf_sc(user_ids, item_ids, su, si, ut, it, wb, bb)
    return out.reshape(B, 1)
